# baseline (speedup 1.0000x reference)
import functools

import jax
import jax.numpy as jnp
from jax import lax
from jax.experimental import pallas as pl
from jax.experimental.pallas import tpu as pltpu
from jax.experimental.pallas import tpu_sc as plsc

D_MODEL = 2048
NUM_EXPERTS = 8
HIDDEN = 1024
B = 4
L = 2048
N_TOKENS = B * L
N_ASSIGN = 2 * N_TOKENS
TILE = 256
N_PAD = N_ASSIGN + NUM_EXPERTS * TILE
N_TILES = N_PAD // TILE

NC = 2
NW = 32
ROWS_PER_W = N_PAD // NW
G_CHUNK = 16
G_NBUF = 3
TOK_PER_W = N_TOKENS // NW
C_CHUNK = 8

def _sc_mesh():
    return plsc.VectorSubcoreMesh(core_axis_name="c", subcore_axis_name="s")


def _routing(x_flat, gate_w, gate_b):
    logits = (jnp.einsum('bld,de->ble', x_flat.reshape(B, L, D_MODEL), gate_w)
              + gate_b)
    lg = logits.reshape(N_TOKENS, NUM_EXPERTS)
    e0 = jnp.argmax(lg, axis=-1).astype(jnp.int32)
    l0 = jnp.take_along_axis(lg, e0[:, None], axis=-1)[:, 0]
    ids = jnp.arange(NUM_EXPERTS, dtype=jnp.int32)[None, :]
    e1 = jnp.argmax(jnp.where(ids == e0[:, None], -jnp.inf, lg),
                    axis=-1).astype(jnp.int32)
    l1 = jnp.take_along_axis(lg, e1[:, None], axis=-1)[:, 0]
    ex = jnp.exp(l1 - l0)
    g0 = 1.0 / (1.0 + ex)
    g1 = ex / (1.0 + ex)

    eids = jnp.stack([e0, e1], axis=-1).reshape(-1)
    gvals = jnp.stack([g0, g1], axis=-1).reshape(-1)

    one_hot = (eids[:, None] ==
               jnp.arange(NUM_EXPERTS, dtype=jnp.int32)[None, :]).astype(jnp.int32)
    cum = jnp.cumsum(one_hot, axis=0)
    rank = jnp.take_along_axis(cum, eids[:, None], axis=1)[:, 0] - 1
    counts = cum[-1]
    padded = ((counts + TILE - 1) // TILE) * TILE
    poff = jnp.concatenate([jnp.zeros((1,), jnp.int32),
                            jnp.cumsum(padded).astype(jnp.int32)])
    dest = poff[eids] + rank

    gate_sorted = jnp.zeros((N_PAD,), jnp.float32).at[dest].set(gvals)
    tile_expert = jnp.clip(
        jnp.searchsorted(poff[1:], jnp.arange(N_TILES, dtype=jnp.int32) * TILE,
                         side='right'),
        0, NUM_EXPERTS - 1).astype(jnp.int32)
    pos0 = dest[0::2]
    pos1 = dest[1::2]
    return gate_sorted, tile_expert, pos0, pos1


def _sc_gather(x_flat, pos0, pos1):
    n_chunks = TOK_PER_W // G_CHUNK
    p0r = pos0.reshape(NW, n_chunks, G_CHUNK)
    p1r = pos1.reshape(NW, n_chunks, G_CHUNK)

    @functools.partial(
        pl.kernel,
        out_type=jax.ShapeDtypeStruct((N_PAD, D_MODEL // 2), jnp.int32),
        mesh=_sc_mesh(),
        scratch_types=[
            pltpu.VMEM((n_chunks, G_CHUNK), jnp.int32),
            pltpu.VMEM((n_chunks, G_CHUNK), jnp.int32),
            [pltpu.VMEM((G_CHUNK, D_MODEL // 2), jnp.int32)] * G_NBUF,
            [pltpu.SemaphoreType.DMA] * G_NBUF,
            [pltpu.SemaphoreType.DMA] * G_NBUF,
            [pltpu.SemaphoreType.DMA] * G_NBUF,
        ],
    )
    def scatter_kernel(x_hbm, p0_hbm, p1_hbm, out_hbm,
                       p0_v, p1_v, rows, sr, s0, s1):
        wid = lax.axis_index("s") * NC + lax.axis_index("c")
        base0 = wid * TOK_PER_W
        pltpu.sync_copy(p0_hbm.at[wid], p0_v)
        pltpu.sync_copy(p1_hbm.at[wid], p1_v)
        rd = [None] * G_NBUF
        wr = [None] * G_NBUF
        for step in range(n_chunks + G_NBUF - 1):
            if step < n_chunks:
                b = step % G_NBUF
                if wr[b] is not None:
                    wr[b][0].wait()
                    wr[b][1].wait()
                rd[b] = pltpu.async_copy(
                    x_hbm.at[pl.ds(base0 + step * G_CHUNK, G_CHUNK)],
                    rows[b], sr[b])
            j = step - (G_NBUF - 1)
            if 0 <= j < n_chunks:
                bj = j % G_NBUF
                rd[bj].wait()
                wr[bj] = (
                    pltpu.async_copy(rows[bj], out_hbm.at[p0_v.at[j]],
                                     s0[bj]),
                    pltpu.async_copy(rows[bj], out_hbm.at[p1_v.at[j]],
                                     s1[bj]),
                )
        for b in range(G_NBUF):
            if wr[b] is not None:
                wr[b][0].wait()
                wr[b][1].wait()

    return scatter_kernel(x_flat, p0r, p1r)


def _ffn_body(s_ref, x_ref, w1_ref, b1_ref, w2_ref, b2_ref, g_ref, out_ref):
    u = x_ref[...]
    lo = lax.bitcast_convert_type(u << 16, jnp.float32)
    hi = lax.bitcast_convert_type(u & jnp.int32(-65536), jnp.float32)
    xb = jnp.concatenate([lo, hi], axis=1).astype(jnp.bfloat16)
    h = jnp.dot(xb, w1_ref[0].astype(jnp.bfloat16),
                preferred_element_type=jnp.float32)
    h = h + b1_ref[0]
    h = 0.5 * h * (1.0 + lax.erf(h * 0.7071067811865476))
    hb = h.astype(jnp.bfloat16)
    y = jnp.dot(hb, w2_ref[0].astype(jnp.bfloat16),
                preferred_element_type=jnp.float32)
    y = y + b2_ref[0]
    out_ref[...] = y * g_ref[...][:, :1]


def _tc_ffn(x_sorted, w1b, b1, w2b, b2, gate2d, tile_expert):
    grid_spec = pltpu.PrefetchScalarGridSpec(
        num_scalar_prefetch=1,
        grid=(N_TILES,),
        in_specs=[
            pl.BlockSpec((TILE, D_MODEL // 2), lambda i, s: (i, 0)),
            pl.BlockSpec((1, D_MODEL, HIDDEN), lambda i, s: (s[i], 0, 0)),
            pl.BlockSpec((1, 1, HIDDEN), lambda i, s: (s[i], 0, 0)),
            pl.BlockSpec((1, HIDDEN, D_MODEL), lambda i, s: (s[i], 0, 0)),
            pl.BlockSpec((1, 1, D_MODEL), lambda i, s: (s[i], 0, 0)),
            pl.BlockSpec((TILE, 128), lambda i, s: (i, 0)),
        ],
        out_specs=pl.BlockSpec((TILE, D_MODEL), lambda i, s: (i, 0)),
    )
    return pl.pallas_call(
        _ffn_body,
        grid_spec=grid_spec,
        out_shape=jax.ShapeDtypeStruct((N_PAD, D_MODEL), jnp.float32),
        compiler_params=pltpu.CompilerParams(
            dimension_semantics=("arbitrary",)),
    )(tile_expert, x_sorted, w1b, b1, w2b, b2, gate2d)


def _sc_combine(y_sorted, pos0, pos1):
    @functools.partial(
        pl.kernel,
        out_type=jax.ShapeDtypeStruct((N_TOKENS, D_MODEL), jnp.float32),
        mesh=_sc_mesh(),
        scratch_types=[
            pltpu.VMEM((TOK_PER_W,), jnp.int32),
            pltpu.VMEM((TOK_PER_W,), jnp.int32),
            pltpu.VMEM((C_CHUNK, D_MODEL), jnp.float32),
            pltpu.VMEM((C_CHUNK, D_MODEL), jnp.float32),
            pltpu.VMEM((C_CHUNK, D_MODEL), jnp.float32),
            pltpu.VMEM((C_CHUNK, D_MODEL), jnp.float32),
            pltpu.SemaphoreType.DMA,
            pltpu.SemaphoreType.DMA,
            pltpu.SemaphoreType.DMA,
            pltpu.SemaphoreType.DMA,
            pltpu.SemaphoreType.DMA,
            pltpu.SemaphoreType.DMA,
        ],
    )
    def combine_kernel(y_hbm, p0_hbm, p1_hbm, out_hbm,
                       i0_all, i1_all, r0a, r1a, r0b, r1b,
                       sa0, sa1, sb0, sb1, swa, swb):
        wid = lax.axis_index("s") * NC + lax.axis_index("c")
        base0 = wid * TOK_PER_W
        n_pairs = TOK_PER_W // (2 * C_CHUNK)
        pltpu.sync_copy(p0_hbm.at[pl.ds(base0, TOK_PER_W)], i0_all)
        pltpu.sync_copy(p1_hbm.at[pl.ds(base0, TOK_PER_W)], i1_all)

        def accum_rows(r0, r1):
            def row(j, c2):
                for sl in range(D_MODEL // 16):
                    plsc.addupdate(r0.at[j, pl.ds(sl * 16, 16)],
                                   r1[j, pl.ds(sl * 16, 16)])
                return c2
            lax.fori_loop(0, C_CHUNK, row, 0)

        def pair(k, carry):
            off_a = (2 * k) * C_CHUNK
            off_b = off_a + C_CHUNK
            ca0 = pltpu.async_copy(
                y_hbm.at[i0_all.at[pl.ds(off_a, C_CHUNK)]], r0a, sa0)
            ca1 = pltpu.async_copy(
                y_hbm.at[i1_all.at[pl.ds(off_a, C_CHUNK)]], r1a, sa1)
            cb0 = pltpu.async_copy(
                y_hbm.at[i0_all.at[pl.ds(off_b, C_CHUNK)]], r0b, sb0)
            cb1 = pltpu.async_copy(
                y_hbm.at[i1_all.at[pl.ds(off_b, C_CHUNK)]], r1b, sb1)
            ca0.wait()
            ca1.wait()
            accum_rows(r0a, r1a)
            wba = pltpu.async_copy(
                r0a, out_hbm.at[pl.ds(base0 + off_a, C_CHUNK)], swa)
            cb0.wait()
            cb1.wait()
            accum_rows(r0b, r1b)
            wbb = pltpu.async_copy(
                r0b, out_hbm.at[pl.ds(base0 + off_b, C_CHUNK)], swb)
            wba.wait()
            wbb.wait()
            return carry

        lax.fori_loop(0, n_pairs, pair, 0)

    return combine_kernel(y_sorted, pos0, pos1)


def kernel(x, gate_w, gate_b, w1, b1, w2, b2):
    x_flat = x.reshape(N_TOKENS, D_MODEL)
    gate_sorted, tile_expert, pos0, pos1 = _routing(x_flat, gate_w, gate_b)

    bits = lax.bitcast_convert_type(x_flat, jnp.uint32)

    def _rne(bv):
        return (bv + jnp.uint32(0x7FFF) + ((bv >> 16) & jnp.uint32(1))) >> 16

    lo = _rne(bits[:, :D_MODEL // 2])
    hi = _rne(bits[:, D_MODEL // 2:])
    x_packed = lax.bitcast_convert_type(lo | (hi << 16), jnp.int32)

    x_sorted = _sc_gather(x_packed, pos0, pos1)

    gate2d = jnp.broadcast_to(gate_sorted[:, None], (N_PAD, 128))
    y_sorted = _tc_ffn(x_sorted, w1, b1.reshape(NUM_EXPERTS, 1, HIDDEN),
                       w2, b2.reshape(NUM_EXPERTS, 1, D_MODEL),
                       gate2d, tile_expert)

    out_flat = _sc_combine(y_sorted, pos0, pos1)
    return out_flat.reshape(B, L, D_MODEL)


# speedup vs baseline: 1.1294x; 1.1294x over previous
import functools

import jax
import jax.numpy as jnp
from jax import lax
from jax.experimental import pallas as pl
from jax.experimental.pallas import tpu as pltpu
from jax.experimental.pallas import tpu_sc as plsc

D_MODEL = 2048
NUM_EXPERTS = 8
HIDDEN = 1024
B = 4
L = 2048
N_TOKENS = B * L
N_ASSIGN = 2 * N_TOKENS
TILE = 256
N_PAD = N_ASSIGN + NUM_EXPERTS * TILE
N_TILES = N_PAD // TILE

NC = 2
NW = 32
ROWS_PER_W = N_PAD // NW
G_CHUNK = 16
G_NBUF = 3
TOK_PER_W = N_TOKENS // NW
C_CHUNK = 8

def _sc_mesh():
    return plsc.VectorSubcoreMesh(core_axis_name="c", subcore_axis_name="s")


def _routing(x_flat, gate_w, gate_b):
    logits = (jnp.einsum('bld,de->ble', x_flat.reshape(B, L, D_MODEL), gate_w)
              + gate_b)
    lg = logits.reshape(N_TOKENS, NUM_EXPERTS)
    e0 = jnp.argmax(lg, axis=-1).astype(jnp.int32)
    l0 = jnp.take_along_axis(lg, e0[:, None], axis=-1)[:, 0]
    ids = jnp.arange(NUM_EXPERTS, dtype=jnp.int32)[None, :]
    e1 = jnp.argmax(jnp.where(ids == e0[:, None], -jnp.inf, lg),
                    axis=-1).astype(jnp.int32)
    l1 = jnp.take_along_axis(lg, e1[:, None], axis=-1)[:, 0]
    ex = jnp.exp(l1 - l0)
    g0 = 1.0 / (1.0 + ex)
    g1 = ex / (1.0 + ex)

    eids = jnp.stack([e0, e1], axis=-1).reshape(-1)
    gvals = jnp.stack([g0, g1], axis=-1).reshape(-1)

    one_hot = (eids[:, None] ==
               jnp.arange(NUM_EXPERTS, dtype=jnp.int32)[None, :]).astype(jnp.int32)
    cum = jnp.cumsum(one_hot, axis=0)
    rank = jnp.take_along_axis(cum, eids[:, None], axis=1)[:, 0] - 1
    counts = cum[-1]
    padded = ((counts + TILE - 1) // TILE) * TILE
    poff = jnp.concatenate([jnp.zeros((1,), jnp.int32),
                            jnp.cumsum(padded).astype(jnp.int32)])
    dest = poff[eids] + rank

    gate_sorted = jnp.zeros((N_PAD,), jnp.float32).at[dest].set(gvals)
    tile_expert = jnp.clip(
        jnp.searchsorted(poff[1:], jnp.arange(N_TILES, dtype=jnp.int32) * TILE,
                         side='right'),
        0, NUM_EXPERTS - 1).astype(jnp.int32)
    pos0 = dest[0::2]
    pos1 = dest[1::2]
    return gate_sorted, tile_expert, pos0, pos1


def _sc_gather(x_flat, pos0, pos1):
    n_chunks = TOK_PER_W // G_CHUNK
    p0r = pos0.reshape(NW, n_chunks, G_CHUNK)
    p1r = pos1.reshape(NW, n_chunks, G_CHUNK)

    @functools.partial(
        pl.kernel,
        out_type=jax.ShapeDtypeStruct((N_PAD, D_MODEL), jnp.float32),
        mesh=_sc_mesh(),
        scratch_types=[
            pltpu.VMEM((n_chunks, G_CHUNK), jnp.int32),
            pltpu.VMEM((n_chunks, G_CHUNK), jnp.int32),
            [pltpu.VMEM((G_CHUNK, D_MODEL), jnp.float32)] * G_NBUF,
            [pltpu.SemaphoreType.DMA] * G_NBUF,
            [pltpu.SemaphoreType.DMA] * G_NBUF,
            [pltpu.SemaphoreType.DMA] * G_NBUF,
        ],
    )
    def scatter_kernel(x_hbm, p0_hbm, p1_hbm, out_hbm,
                       p0_v, p1_v, rows, sr, s0, s1):
        wid = lax.axis_index("s") * NC + lax.axis_index("c")
        base0 = wid * TOK_PER_W
        pltpu.sync_copy(p0_hbm.at[wid], p0_v)
        pltpu.sync_copy(p1_hbm.at[wid], p1_v)
        rd = [None] * G_NBUF
        wr = [None] * G_NBUF
        for step in range(n_chunks + G_NBUF - 1):
            if step < n_chunks:
                b = step % G_NBUF
                if wr[b] is not None:
                    wr[b][0].wait()
                    wr[b][1].wait()
                rd[b] = pltpu.async_copy(
                    x_hbm.at[pl.ds(base0 + step * G_CHUNK, G_CHUNK)],
                    rows[b], sr[b])
            j = step - (G_NBUF - 1)
            if 0 <= j < n_chunks:
                bj = j % G_NBUF
                rd[bj].wait()
                wr[bj] = (
                    pltpu.async_copy(rows[bj], out_hbm.at[p0_v.at[j]],
                                     s0[bj]),
                    pltpu.async_copy(rows[bj], out_hbm.at[p1_v.at[j]],
                                     s1[bj]),
                )
        for b in range(G_NBUF):
            if wr[b] is not None:
                wr[b][0].wait()
                wr[b][1].wait()

    return scatter_kernel(x_flat, p0r, p1r)


def _ffn_body(s_ref, x_ref, w1_ref, b1_ref, w2_ref, b2_ref, g_ref, out_ref):
    xb = x_ref[...].astype(jnp.bfloat16)
    h = jnp.dot(xb, w1_ref[0].astype(jnp.bfloat16),
                preferred_element_type=jnp.float32)
    h = h + b1_ref[0]
    h = 0.5 * h * (1.0 + lax.erf(h * 0.7071067811865476))
    hb = h.astype(jnp.bfloat16)
    y = jnp.dot(hb, w2_ref[0].astype(jnp.bfloat16),
                preferred_element_type=jnp.float32)
    y = y + b2_ref[0]
    out_ref[...] = y * g_ref[...][:, :1]


def _tc_ffn(x_sorted, w1b, b1, w2b, b2, gate2d, tile_expert):
    grid_spec = pltpu.PrefetchScalarGridSpec(
        num_scalar_prefetch=1,
        grid=(N_TILES,),
        in_specs=[
            pl.BlockSpec((TILE, D_MODEL), lambda i, s: (i, 0)),
            pl.BlockSpec((1, D_MODEL, HIDDEN), lambda i, s: (s[i], 0, 0)),
            pl.BlockSpec((1, 1, HIDDEN), lambda i, s: (s[i], 0, 0)),
            pl.BlockSpec((1, HIDDEN, D_MODEL), lambda i, s: (s[i], 0, 0)),
            pl.BlockSpec((1, 1, D_MODEL), lambda i, s: (s[i], 0, 0)),
            pl.BlockSpec((TILE, 128), lambda i, s: (i, 0)),
        ],
        out_specs=pl.BlockSpec((TILE, D_MODEL), lambda i, s: (i, 0)),
    )
    return pl.pallas_call(
        _ffn_body,
        grid_spec=grid_spec,
        out_shape=jax.ShapeDtypeStruct((N_PAD, D_MODEL), jnp.float32),
        compiler_params=pltpu.CompilerParams(
            dimension_semantics=("arbitrary",)),
    )(tile_expert, x_sorted, w1b, b1, w2b, b2, gate2d)


def _sc_combine(y_sorted, pos0, pos1):
    @functools.partial(
        pl.kernel,
        out_type=jax.ShapeDtypeStruct((N_TOKENS, D_MODEL), jnp.float32),
        mesh=_sc_mesh(),
        scratch_types=[
            pltpu.VMEM((TOK_PER_W,), jnp.int32),
            pltpu.VMEM((TOK_PER_W,), jnp.int32),
            pltpu.VMEM((C_CHUNK, D_MODEL), jnp.float32),
            pltpu.VMEM((C_CHUNK, D_MODEL), jnp.float32),
            pltpu.VMEM((C_CHUNK, D_MODEL), jnp.float32),
            pltpu.VMEM((C_CHUNK, D_MODEL), jnp.float32),
            pltpu.SemaphoreType.DMA,
            pltpu.SemaphoreType.DMA,
            pltpu.SemaphoreType.DMA,
            pltpu.SemaphoreType.DMA,
            pltpu.SemaphoreType.DMA,
            pltpu.SemaphoreType.DMA,
        ],
    )
    def combine_kernel(y_hbm, p0_hbm, p1_hbm, out_hbm,
                       i0_all, i1_all, r0a, r1a, r0b, r1b,
                       sa0, sa1, sb0, sb1, swa, swb):
        wid = lax.axis_index("s") * NC + lax.axis_index("c")
        base0 = wid * TOK_PER_W
        n_pairs = TOK_PER_W // (2 * C_CHUNK)
        pltpu.sync_copy(p0_hbm.at[pl.ds(base0, TOK_PER_W)], i0_all)
        pltpu.sync_copy(p1_hbm.at[pl.ds(base0, TOK_PER_W)], i1_all)

        def accum_rows(r0, r1):
            def row(j, c2):
                for sl in range(D_MODEL // 16):
                    plsc.addupdate(r0.at[j, pl.ds(sl * 16, 16)],
                                   r1[j, pl.ds(sl * 16, 16)])
                return c2
            lax.fori_loop(0, C_CHUNK, row, 0)

        def pair(k, carry):
            off_a = (2 * k) * C_CHUNK
            off_b = off_a + C_CHUNK
            ca0 = pltpu.async_copy(
                y_hbm.at[i0_all.at[pl.ds(off_a, C_CHUNK)]], r0a, sa0)
            ca1 = pltpu.async_copy(
                y_hbm.at[i1_all.at[pl.ds(off_a, C_CHUNK)]], r1a, sa1)
            cb0 = pltpu.async_copy(
                y_hbm.at[i0_all.at[pl.ds(off_b, C_CHUNK)]], r0b, sb0)
            cb1 = pltpu.async_copy(
                y_hbm.at[i1_all.at[pl.ds(off_b, C_CHUNK)]], r1b, sb1)
            ca0.wait()
            ca1.wait()
            accum_rows(r0a, r1a)
            wba = pltpu.async_copy(
                r0a, out_hbm.at[pl.ds(base0 + off_a, C_CHUNK)], swa)
            cb0.wait()
            cb1.wait()
            accum_rows(r0b, r1b)
            wbb = pltpu.async_copy(
                r0b, out_hbm.at[pl.ds(base0 + off_b, C_CHUNK)], swb)
            wba.wait()
            wbb.wait()
            return carry

        lax.fori_loop(0, n_pairs, pair, 0)

    return combine_kernel(y_sorted, pos0, pos1)


def kernel(x, gate_w, gate_b, w1, b1, w2, b2):
    x_flat = x.reshape(N_TOKENS, D_MODEL)
    gate_sorted, tile_expert, pos0, pos1 = _routing(x_flat, gate_w, gate_b)

    x_sorted = _sc_gather(x_flat, pos0, pos1)

    gate2d = jnp.broadcast_to(gate_sorted[:, None], (N_PAD, 128))
    y_sorted = _tc_ffn(x_sorted, w1, b1.reshape(NUM_EXPERTS, 1, HIDDEN),
                       w2, b2.reshape(NUM_EXPERTS, 1, D_MODEL),
                       gate2d, tile_expert)

    out_flat = _sc_combine(y_sorted, pos0, pos1)
    return out_flat.reshape(B, L, D_MODEL)
